# Initial kernel scaffold; baseline (speedup 1.0000x reference)
import jax
import jax.numpy as jnp
from jax.experimental import pallas as pl


def kernel(x, indices, output_size, stride, padding, out):
    raise NotImplementedError("write your pallas kernel here")


# speedup vs baseline: 4.7218x; 4.7218x over previous
import functools

import jax
import jax.numpy as jnp
from jax import lax
from jax.experimental import pallas as pl
from jax.experimental.pallas import tpu as pltpu
from jax.experimental.pallas import tpu_sc as plsc

NPLANES = 192
PLANE = 16 * 128 * 128
NIN = 8 * 64 * 64
NW = 32
PPW = NPLANES // NW
ROWS = NIN // 128
ZCH = 16384
ZPP = PLANE // ZCH

_mesh = plsc.VectorSubcoreMesh(core_axis_name="c", subcore_axis_name="s")


@functools.partial(
    pl.kernel,
    out_type=jax.ShapeDtypeStruct((NPLANES * PLANE,), jnp.float32),
    mesh=_mesh,
    scratch_types=[
        pltpu.VMEM((ZCH,), jnp.float32),
        pltpu.VMEM((NIN,), jnp.int32),
        pltpu.VMEM((NIN,), jnp.float32),
        pltpu.VMEM((NIN,), jnp.int32),
        pltpu.SemaphoreType.DMA,
        pltpu.SemaphoreType.DMA,
    ],
)
def _unpool_sc(x_hbm, idx_hbm, out_hbm, zbuf, ibuf, vbuf, gbuf, zsem, ssem):
    wid = lax.axis_index("s") * 2 + lax.axis_index("c")
    base0 = wid * PPW * PLANE

    zv = jnp.zeros((16,), jnp.float32)

    def _zinit(i, _):
        zbuf[pl.ds(i * 16, 16)] = zv
        return 0

    lax.fori_loop(0, ZCH // 16, _zinit, 0)

    def _zfire(j, _):
        pltpu.async_copy(zbuf, out_hbm.at[pl.ds(base0 + j * ZCH, ZCH)], zsem)
        return 0

    lax.fori_loop(0, PPW * ZPP, _zfire, 0)

    def _zdrain(j, _):
        pltpu.make_async_copy(
            zbuf, out_hbm.at[pl.ds(base0 + j * ZCH, ZCH)], zsem
        ).wait()
        return 0

    lax.fori_loop(0, PPW * ZPP, _zdrain, 0)

    def _plane(p_i, _):
        p = wid * PPW + p_i
        pltpu.sync_copy(idx_hbm.at[p], ibuf)
        pltpu.sync_copy(x_hbm.at[p], vbuf)
        pbase = p * PLANE

        def _add(r, _):
            for j in range(8):
                sl = pl.ds(r * 128 + j * 16, 16)
                gbuf[sl] = ibuf[sl] + pbase
            return 0

        lax.fori_loop(0, ROWS, _add, 0)

        pltpu.async_copy(vbuf, out_hbm.at[gbuf], ssem).wait()
        return 0

    lax.fori_loop(0, PPW, _plane, 0)


def kernel(x, indices, output_size, stride, padding, out):
    xr = x.reshape(NPLANES, NIN)
    ir = indices.reshape(NPLANES, NIN)
    flat = _unpool_sc(xr, ir)
    return flat.reshape(out.shape)
